# baseline (speedup 1.0000x reference)
import functools

import jax
import jax.numpy as jnp
from jax import lax
from jax.experimental import pallas as pl
from jax.experimental.pallas import tpu as pltpu
from jax.experimental.pallas import tpu_sc as plsc

_TN = 2048


def _nnd_body(x_ref, yt_ref, d1_ref, d2_ref):
    i = pl.program_id(1)
    x = x_ref[0]
    y = yt_ref[0]
    f32, bf16 = jnp.float32, jnp.bfloat16
    tn = x.shape[0]
    m = y.shape[1]
    x2 = (x[:, 0:1] * x[:, 0:1] + x[:, 1:2] * x[:, 1:2]
          + x[:, 2:3] * x[:, 2:3])
    y2 = (y[0:1, :] * y[0:1, :] + y[1:2, :] * y[1:2, :]
          + y[2:3, :] * y[2:3, :])
    x2h = x2.astype(bf16)
    x2l = (x2 - x2h.astype(f32)).astype(bf16)
    y2h = y2.astype(bf16)
    y2l = (y2 - y2h.astype(f32)).astype(bf16)
    one_c = jnp.ones((tn, 1), bf16)
    one_r = jnp.ones((1, m), bf16)
    xa = jnp.concatenate(
        [(x * -2.0).astype(bf16), x2h, x2l, one_c, one_c], axis=1)
    ya = jnp.concatenate(
        [y.astype(bf16), one_r, one_r, y2h, y2l], axis=0)
    nchunks = 4
    mc = m // nchunks
    d1p = []
    m2p = []
    for ci in range(nchunks):
        d = jax.lax.dot_general(xa, ya[:, ci * mc:(ci + 1) * mc],
                                (((1,), (0,)), ((), ())),
                                preferred_element_type=jnp.float32)
        d1p.append(jnp.min(d, axis=1, keepdims=True))
        m2p.append(jnp.min(d, axis=0, keepdims=True))
    d1c = d1p[0]
    for ci in range(1, nchunks):
        d1c = jnp.minimum(d1c, d1p[ci])
    d1_ref[0] = d1c
    m2 = jnp.concatenate(m2p, axis=1)

    @pl.when(i == 0)
    def _init():
        d2_ref[0] = m2

    @pl.when(i > 0)
    def _acc():
        d2_ref[0] = jnp.minimum(d2_ref[0], m2)


def _nnd_tc(x, y):
    B, N, _ = x.shape
    M = y.shape[1]
    yt = jnp.transpose(y, (0, 2, 1))
    d1, d2 = pl.pallas_call(
        _nnd_body,
        grid=(B, N // _TN),
        in_specs=[
            pl.BlockSpec((1, _TN, 3), lambda b, i: (b, i, 0)),
            pl.BlockSpec((1, 3, M), lambda b, i: (b, 0, 0)),
        ],
        out_specs=[
            pl.BlockSpec((1, _TN, 1), lambda b, i: (b, i, 0)),
            pl.BlockSpec((1, 1, M), lambda b, i: (b, 0, 0)),
        ],
        out_shape=[
            jax.ShapeDtypeStruct((B, N, 1), jnp.float32),
            jax.ShapeDtypeStruct((B, 1, M), jnp.float32),
        ],
    )(x, yt)
    return d1[:, :, 0], d2[:, 0, :]


_NC, _NS, _L = 2, 16, 16
_NBLK = 4
_UNROLL = 4


def _bf16_round(v):
    u = lax.bitcast_convert_type(v, jnp.uint32)
    r = (u + ((u >> 16) & jnp.uint32(1)) + jnp.uint32(0x7FFF)) \
        & jnp.uint32(0xFFFF0000)
    return lax.bitcast_convert_type(r, jnp.float32)


def _gather16(v, idx):
    return lax.gather(
        v, idx[:, None],
        lax.GatherDimensionNumbers(offset_dims=(), collapsed_slice_dims=(0,),
                                   start_index_map=(0,)),
        slice_sizes=(1,),
        mode=lax.GatherScatterMode.PROMISE_IN_BOUNDS)


def _sc_compute_rows(xb_v, y_v, acc2_v, d1_v, n_chunk, M):
    f32 = jnp.float32

    def n_block16(i, _):
        nb16 = i * _L
        sl16 = pl.ds(nb16, _L)
        a0v = xb_v[0, sl16]
        a1v = xb_v[1, sl16]
        a2v = xb_v[2, sl16]
        sxv = xb_v[3, sl16]
        lane = lax.iota(jnp.int32, _L)
        dv = jnp.full((_L,), 0.0, f32)
        for r in range(_L // _NBLK):
            coef = [(a0v[_NBLK * r + j], a1v[_NBLK * r + j],
                     a2v[_NBLK * r + j], sxv[_NBLK * r + j])
                    for j in range(_NBLK)]

            inf16 = jnp.full((_L,), jnp.inf, f32)

            @plsc.parallel_loop(0, M // _L, _UNROLL,
                                carry=tuple(inf16 for _ in range(_NBLK)))
            def accs(mj, accs_c):
                accs_c = list(accs_c)
                sls = [pl.ds((mj + u) * _L, _L) for u in range(_UNROLL)]
                ybs = [(y_v[0, sl], y_v[1, sl], y_v[2, sl], y_v[3, sl])
                       for sl in sls]
                a2s = [acc2_v[sl] for sl in sls]
                prods = [[(ybs[u][0] * coef[j][0],
                           ybs[u][1] * coef[j][1],
                           ybs[u][2] * coef[j][2])
                          for j in range(_NBLK)]
                         for u in range(_UNROLL)]
                ts = [[(ybs[u][3] + prods[u][j][2])
                       + (prods[u][j][0] + prods[u][j][1])
                       for j in range(_NBLK)]
                      for u in range(_UNROLL)]
                for j in range(_NBLK):
                    red = accs_c[j]
                    for u in range(_UNROLL):
                        red = jnp.minimum(red, ts[u][j])
                    accs_c[j] = red
                for u in range(_UNROLL):
                    dmins = [ts[u][j] + coef[j][3] for j in range(_NBLK)]
                    while len(dmins) > 1:
                        dmins = [jnp.minimum(dmins[k], dmins[k + 1])
                                 for k in range(0, len(dmins) - 1, 2)] \
                            + ([dmins[-1]] if len(dmins) % 2 else [])
                    acc2_v[sls[u]] = jnp.minimum(a2s[u], dmins[0])
                return tuple(accs_c)

            for j in range(_NBLK):
                red = accs[j]
                for sh in (8, 4, 2, 1):
                    pidx = jnp.bitwise_xor(lane, sh)
                    red = jnp.minimum(red, _gather16(red, pidx))
                dv = jnp.where(lane == (_NBLK * r + j), red, dv)
        d1_v[sl16] = dv + sxv
        return 0

    lax.fori_loop(0, n_chunk // _L, n_block16, 0)


def _make_sc(B, N, M):
    bps = B // _NC
    wpb = _NS // bps
    n_chunk = N // wpb
    mseg = M // wpb
    f32 = jnp.float32
    mesh = plsc.VectorSubcoreMesh(core_axis_name="c", subcore_axis_name="s",
                                  num_cores=_NC, num_subcores=_NS)

    @functools.partial(
        pl.kernel, mesh=mesh,
        out_type=[jax.ShapeDtypeStruct((B, N), f32),
                  jax.ShapeDtypeStruct((B, M), f32)],
        scratch_types=[
            pltpu.VMEM((4, n_chunk), f32),
            pltpu.VMEM((4, M), f32),
            pltpu.VMEM((3, n_chunk), f32),
            pltpu.VMEM((3, M), f32),
            pltpu.VMEM((M,), f32),
            pltpu.VMEM((n_chunk,), f32),
            pltpu.VMEM((wpb, mseg), f32),
            pltpu.VMEM_SHARED((_NS, M), f32),
        ],
    )
    def sc_kernel(xt, yt, d1o, d2o, xb_v, y_v, xtmp_v, ytmp_v,
                  acc2_v, d1_v, mrg_v, sh_sp):
        c = lax.axis_index("c")
        s = lax.axis_index("s")
        b = c * bps + s // wpb
        q = s % wpb
        n0 = q * n_chunk

        pltpu.sync_copy(xt.at[b, :, pl.ds(n0, n_chunk)], xtmp_v)
        pltpu.sync_copy(yt.at[b], ytmp_v)

        def build_x(i, _):
            sl = pl.ds(i * _L, _L)
            x0 = xtmp_v[0, sl]
            x1 = xtmp_v[1, sl]
            x2c = xtmp_v[2, sl]
            xb_v[0, sl] = _bf16_round(x0) * -2.0
            xb_v[1, sl] = _bf16_round(x1) * -2.0
            xb_v[2, sl] = _bf16_round(x2c) * -2.0
            xb_v[3, sl] = x0 * x0 + x1 * x1 + x2c * x2c
            return 0

        lax.fori_loop(0, n_chunk // _L, build_x, 0)

        def build_y(i, _):
            sl = pl.ds(i * _L, _L)
            y0 = ytmp_v[0, sl]
            y1 = ytmp_v[1, sl]
            y2c = ytmp_v[2, sl]
            y_v[0, sl] = _bf16_round(y0)
            y_v[1, sl] = _bf16_round(y1)
            y_v[2, sl] = _bf16_round(y2c)
            y_v[3, sl] = y0 * y0 + y1 * y1 + y2c * y2c
            acc2_v[sl] = jnp.full((_L,), jnp.inf, f32)
            return 0

        lax.fori_loop(0, M // _L, build_y, 0)

        _sc_compute_rows(xb_v, y_v, acc2_v, d1_v, n_chunk, M)

        pltpu.sync_copy(d1_v, d1o.at[b, pl.ds(n0, n_chunk)])

        pltpu.sync_copy(acc2_v, sh_sp.at[s])
        plsc.subcore_barrier()
        g0 = (s // wpb) * wpb
        m0 = q * mseg
        for k in range(wpb):
            pltpu.sync_copy(sh_sp.at[g0 + k, pl.ds(m0, mseg)], mrg_v.at[k])

        def mrg(i, _):
            sl = pl.ds(i * _L, _L)
            r = mrg_v[0, sl]
            for k in range(1, wpb):
                r = jnp.minimum(r, mrg_v[k, sl])
            acc2_v[sl] = r
            return 0

        lax.fori_loop(0, mseg // _L, mrg, 0)
        pltpu.sync_copy(acc2_v.at[pl.ds(0, mseg)], d2o.at[b, pl.ds(m0, mseg)])

    return sc_kernel


def _make_sc1(N, M):
    n_chunk = N // (_NC * _NS)
    mseg = M // _NS
    f32 = jnp.float32
    mesh = plsc.VectorSubcoreMesh(core_axis_name="c", subcore_axis_name="s",
                                  num_cores=_NC, num_subcores=_NS)

    @functools.partial(
        pl.kernel, mesh=mesh,
        out_type=[jax.ShapeDtypeStruct((1, N), f32),
                  jax.ShapeDtypeStruct((_NC, M), f32)],
        scratch_types=[
            pltpu.VMEM((4, n_chunk), f32),
            pltpu.VMEM((4, M), f32),
            pltpu.VMEM((3, n_chunk), f32),
            pltpu.VMEM((3, M), f32),
            pltpu.VMEM((M,), f32),
            pltpu.VMEM((n_chunk,), f32),
            pltpu.VMEM((_NS, mseg), f32),
            pltpu.VMEM_SHARED((_NS, M), f32),
        ],
    )
    def sc_kernel(xt, yt, d1o, d2p, xb_v, y_v, xtmp_v, ytmp_v,
                  acc2_v, d1_v, mrg_v, sh_sp):
        c = lax.axis_index("c")
        s = lax.axis_index("s")
        q32 = s * _NC + c
        n0 = q32 * n_chunk

        pltpu.sync_copy(xt.at[0, :, pl.ds(n0, n_chunk)], xtmp_v)
        pltpu.sync_copy(yt.at[0], ytmp_v)

        def build_x(i, _):
            sl = pl.ds(i * _L, _L)
            x0 = xtmp_v[0, sl]
            x1 = xtmp_v[1, sl]
            x2c = xtmp_v[2, sl]
            xb_v[0, sl] = _bf16_round(x0) * -2.0
            xb_v[1, sl] = _bf16_round(x1) * -2.0
            xb_v[2, sl] = _bf16_round(x2c) * -2.0
            xb_v[3, sl] = x0 * x0 + x1 * x1 + x2c * x2c
            return 0

        lax.fori_loop(0, n_chunk // _L, build_x, 0)

        def build_y(i, _):
            sl = pl.ds(i * _L, _L)
            y0 = ytmp_v[0, sl]
            y1 = ytmp_v[1, sl]
            y2c = ytmp_v[2, sl]
            y_v[0, sl] = _bf16_round(y0)
            y_v[1, sl] = _bf16_round(y1)
            y_v[2, sl] = _bf16_round(y2c)
            y_v[3, sl] = y0 * y0 + y1 * y1 + y2c * y2c
            acc2_v[sl] = jnp.full((_L,), jnp.inf, f32)
            return 0

        lax.fori_loop(0, M // _L, build_y, 0)

        _sc_compute_rows(xb_v, y_v, acc2_v, d1_v, n_chunk, M)

        pltpu.sync_copy(d1_v, d1o.at[0, pl.ds(n0, n_chunk)])

        pltpu.sync_copy(acc2_v, sh_sp.at[s])
        plsc.subcore_barrier()
        m0 = s * mseg
        for k in range(_NS):
            pltpu.sync_copy(sh_sp.at[k, pl.ds(m0, mseg)], mrg_v.at[k])

        def mrg(i, _):
            sl = pl.ds(i * _L, _L)
            r = mrg_v[0, sl]
            for k in range(1, _NS):
                r = jnp.minimum(r, mrg_v[k, sl])
            acc2_v[sl] = r
            return 0

        lax.fori_loop(0, mseg // _L, mrg, 0)
        pltpu.sync_copy(acc2_v.at[pl.ds(0, mseg)],
                        d2p.at[c, pl.ds(m0, mseg)])

    return sc_kernel


def _make_sc1_merge(M):
    seg = M // (_NC * _NS)
    f32 = jnp.float32
    mesh = plsc.VectorSubcoreMesh(core_axis_name="c", subcore_axis_name="s",
                                  num_cores=_NC, num_subcores=_NS)

    @functools.partial(
        pl.kernel, mesh=mesh,
        out_type=jax.ShapeDtypeStruct((1, M), f32),
        scratch_types=[
            pltpu.VMEM((seg,), f32),
            pltpu.VMEM((seg,), f32),
        ],
    )
    def merge_kernel(d2p, out, a_v, b_v):
        c = lax.axis_index("c")
        s = lax.axis_index("s")
        m0 = (s * _NC + c) * seg
        pltpu.sync_copy(d2p.at[0, pl.ds(m0, seg)], a_v)
        pltpu.sync_copy(d2p.at[1, pl.ds(m0, seg)], b_v)

        def mrg(i, _):
            sl = pl.ds(i * _L, _L)
            a_v[sl] = jnp.minimum(a_v[sl], b_v[sl])
            return 0

        lax.fori_loop(0, seg // _L, mrg, 0)
        pltpu.sync_copy(a_v, out.at[0, pl.ds(m0, seg)])

    return merge_kernel


def _nnd_sc(x, y):
    B, N, _ = x.shape
    M = y.shape[1]
    xt = jnp.transpose(x, (0, 2, 1))
    yt = jnp.transpose(y, (0, 2, 1))
    if B == 1:
        d1, d2p = _make_sc1(N, M)(xt, yt)
        d2 = _make_sc1_merge(M)(d2p)
        return d1, d2
    d1, d2 = _make_sc(B, N, M)(xt, yt)
    return d1, d2


_B_SC = 1


def kernel(input1, input2):
    B = input1.shape[0]
    btc = B - _B_SC
    outs = []
    if btc > 0:
        outs.append(_nnd_tc(input1[:btc], input2[:btc]))
    if _B_SC > 0:
        outs.append(_nnd_sc(input1[btc:], input2[btc:]))
    if len(outs) == 1:
        return outs[0]
    return (jnp.concatenate([outs[0][0], outs[1][0]], axis=0),
            jnp.concatenate([outs[0][1], outs[1][1]], axis=0))


# speedup vs baseline: 1.8136x; 1.8136x over previous
import functools

import jax
import jax.numpy as jnp
from jax import lax
from jax.experimental import pallas as pl
from jax.experimental.pallas import tpu as pltpu
from jax.experimental.pallas import tpu_sc as plsc

_TN = 2048


def _nnd_body(x_ref, yt_ref, d1_ref, d2_ref):
    i = pl.program_id(1)
    x = x_ref[0]
    y = yt_ref[0]
    f32, bf16 = jnp.float32, jnp.bfloat16
    tn = x.shape[0]
    m = y.shape[1]
    x2 = (x[:, 0:1] * x[:, 0:1] + x[:, 1:2] * x[:, 1:2]
          + x[:, 2:3] * x[:, 2:3])
    y2 = (y[0:1, :] * y[0:1, :] + y[1:2, :] * y[1:2, :]
          + y[2:3, :] * y[2:3, :])
    x2h = x2.astype(bf16)
    x2l = (x2 - x2h.astype(f32)).astype(bf16)
    y2h = y2.astype(bf16)
    y2l = (y2 - y2h.astype(f32)).astype(bf16)
    one_c = jnp.ones((tn, 1), bf16)
    one_r = jnp.ones((1, m), bf16)
    xa = jnp.concatenate(
        [(x * -2.0).astype(bf16), x2h, x2l, one_c, one_c], axis=1)
    ya = jnp.concatenate(
        [y.astype(bf16), one_r, one_r, y2h, y2l], axis=0)
    nchunks = 4
    mc = m // nchunks
    d1p = []
    m2p = []
    for ci in range(nchunks):
        d = jax.lax.dot_general(xa, ya[:, ci * mc:(ci + 1) * mc],
                                (((1,), (0,)), ((), ())),
                                preferred_element_type=jnp.float32)
        d1p.append(jnp.min(d, axis=1, keepdims=True))
        m2p.append(jnp.min(d, axis=0, keepdims=True))
    d1c = d1p[0]
    for ci in range(1, nchunks):
        d1c = jnp.minimum(d1c, d1p[ci])
    d1_ref[0] = d1c
    m2 = jnp.concatenate(m2p, axis=1)

    @pl.when(i == 0)
    def _init():
        d2_ref[0] = m2

    @pl.when(i > 0)
    def _acc():
        d2_ref[0] = jnp.minimum(d2_ref[0], m2)


def _nnd_tc(x, y):
    B, N, _ = x.shape
    M = y.shape[1]
    yt = jnp.transpose(y, (0, 2, 1))
    d1, d2 = pl.pallas_call(
        _nnd_body,
        grid=(B, N // _TN),
        in_specs=[
            pl.BlockSpec((1, _TN, 3), lambda b, i: (b, i, 0)),
            pl.BlockSpec((1, 3, M), lambda b, i: (b, 0, 0)),
        ],
        out_specs=[
            pl.BlockSpec((1, _TN, 1), lambda b, i: (b, i, 0)),
            pl.BlockSpec((1, 1, M), lambda b, i: (b, 0, 0)),
        ],
        out_shape=[
            jax.ShapeDtypeStruct((B, N, 1), jnp.float32),
            jax.ShapeDtypeStruct((B, 1, M), jnp.float32),
        ],
    )(x, yt)
    return d1[:, :, 0], d2[:, 0, :]


_NC, _NS, _L = 2, 16, 16
_NBLK = 4
_UNROLL = 4


def _bf16_round(v):
    u = lax.bitcast_convert_type(v, jnp.uint32)
    r = (u + ((u >> 16) & jnp.uint32(1)) + jnp.uint32(0x7FFF)) \
        & jnp.uint32(0xFFFF0000)
    return lax.bitcast_convert_type(r, jnp.float32)


def _gather16(v, idx):
    return lax.gather(
        v, idx[:, None],
        lax.GatherDimensionNumbers(offset_dims=(), collapsed_slice_dims=(0,),
                                   start_index_map=(0,)),
        slice_sizes=(1,),
        mode=lax.GatherScatterMode.PROMISE_IN_BOUNDS)


def _sc_compute_rows(xb_v, y_v, acc2_v, d1_v, n_chunk, M):
    f32 = jnp.float32

    def n_block16(i, _):
        nb16 = i * _L
        sl16 = pl.ds(nb16, _L)
        a0v = xb_v[0, sl16]
        a1v = xb_v[1, sl16]
        a2v = xb_v[2, sl16]
        sxv = xb_v[3, sl16]
        lane = lax.iota(jnp.int32, _L)
        dv = jnp.full((_L,), 0.0, f32)
        for r in range(_L // _NBLK):
            coef = [(a0v[_NBLK * r + j], a1v[_NBLK * r + j],
                     a2v[_NBLK * r + j], sxv[_NBLK * r + j])
                    for j in range(_NBLK)]

            inf16 = jnp.full((_L,), jnp.inf, f32)

            @plsc.parallel_loop(0, M // _L, _UNROLL,
                                carry=tuple(inf16 for _ in range(_NBLK)))
            def accs(mj, accs_c):
                accs_c = list(accs_c)
                sls = [pl.ds((mj + u) * _L, _L) for u in range(_UNROLL)]
                ybs = [(y_v[0, sl], y_v[1, sl], y_v[2, sl], y_v[3, sl])
                       for sl in sls]
                a2s = [acc2_v[sl] for sl in sls]
                prods = [[(ybs[u][0] * coef[j][0],
                           ybs[u][1] * coef[j][1],
                           ybs[u][2] * coef[j][2])
                          for j in range(_NBLK)]
                         for u in range(_UNROLL)]
                ts = [[(ybs[u][3] + prods[u][j][2])
                       + (prods[u][j][0] + prods[u][j][1])
                       for j in range(_NBLK)]
                      for u in range(_UNROLL)]
                for j in range(_NBLK):
                    red = accs_c[j]
                    for u in range(_UNROLL):
                        red = jnp.minimum(red, ts[u][j])
                    accs_c[j] = red
                for u in range(_UNROLL):
                    dmins = [ts[u][j] + coef[j][3] for j in range(_NBLK)]
                    while len(dmins) > 1:
                        dmins = [jnp.minimum(dmins[k], dmins[k + 1])
                                 for k in range(0, len(dmins) - 1, 2)] \
                            + ([dmins[-1]] if len(dmins) % 2 else [])
                    acc2_v[sls[u]] = jnp.minimum(a2s[u], dmins[0])
                return tuple(accs_c)

            for j in range(_NBLK):
                red = accs[j]
                for sh in (8, 4, 2, 1):
                    pidx = jnp.bitwise_xor(lane, sh)
                    red = jnp.minimum(red, _gather16(red, pidx))
                dv = jnp.where(lane == (_NBLK * r + j), red, dv)
        d1_v[sl16] = dv + sxv
        return 0

    lax.fori_loop(0, n_chunk // _L, n_block16, 0)


def _make_sc(B, N, M):
    bps = B // _NC
    wpb = _NS // bps
    n_chunk = N // wpb
    mseg = M // wpb
    f32 = jnp.float32
    mesh = plsc.VectorSubcoreMesh(core_axis_name="c", subcore_axis_name="s",
                                  num_cores=_NC, num_subcores=_NS)

    @functools.partial(
        pl.kernel, mesh=mesh,
        out_type=[jax.ShapeDtypeStruct((B, N), f32),
                  jax.ShapeDtypeStruct((B, M), f32)],
        scratch_types=[
            pltpu.VMEM((4, n_chunk), f32),
            pltpu.VMEM((4, M), f32),
            pltpu.VMEM((3, n_chunk), f32),
            pltpu.VMEM((3, M), f32),
            pltpu.VMEM((M,), f32),
            pltpu.VMEM((n_chunk,), f32),
            pltpu.VMEM((wpb, mseg), f32),
            pltpu.VMEM_SHARED((_NS, M), f32),
        ],
    )
    def sc_kernel(xt, yt, d1o, d2o, xb_v, y_v, xtmp_v, ytmp_v,
                  acc2_v, d1_v, mrg_v, sh_sp):
        c = lax.axis_index("c")
        s = lax.axis_index("s")
        b = c * bps + s // wpb
        q = s % wpb
        n0 = q * n_chunk

        pltpu.sync_copy(xt.at[b, :, pl.ds(n0, n_chunk)], xtmp_v)
        pltpu.sync_copy(yt.at[b], ytmp_v)

        def build_x(i, _):
            sl = pl.ds(i * _L, _L)
            x0 = xtmp_v[0, sl]
            x1 = xtmp_v[1, sl]
            x2c = xtmp_v[2, sl]
            xb_v[0, sl] = _bf16_round(x0) * -2.0
            xb_v[1, sl] = _bf16_round(x1) * -2.0
            xb_v[2, sl] = _bf16_round(x2c) * -2.0
            xb_v[3, sl] = x0 * x0 + x1 * x1 + x2c * x2c
            return 0

        lax.fori_loop(0, n_chunk // _L, build_x, 0)

        def build_y(i, _):
            sl = pl.ds(i * _L, _L)
            y0 = ytmp_v[0, sl]
            y1 = ytmp_v[1, sl]
            y2c = ytmp_v[2, sl]
            y_v[0, sl] = _bf16_round(y0)
            y_v[1, sl] = _bf16_round(y1)
            y_v[2, sl] = _bf16_round(y2c)
            y_v[3, sl] = y0 * y0 + y1 * y1 + y2c * y2c
            acc2_v[sl] = jnp.full((_L,), jnp.inf, f32)
            return 0

        lax.fori_loop(0, M // _L, build_y, 0)

        _sc_compute_rows(xb_v, y_v, acc2_v, d1_v, n_chunk, M)

        pltpu.sync_copy(d1_v, d1o.at[b, pl.ds(n0, n_chunk)])

        pltpu.sync_copy(acc2_v, sh_sp.at[s])
        plsc.subcore_barrier()
        g0 = (s // wpb) * wpb
        m0 = q * mseg
        for k in range(wpb):
            pltpu.sync_copy(sh_sp.at[g0 + k, pl.ds(m0, mseg)], mrg_v.at[k])

        def mrg(i, _):
            sl = pl.ds(i * _L, _L)
            r = mrg_v[0, sl]
            for k in range(1, wpb):
                r = jnp.minimum(r, mrg_v[k, sl])
            acc2_v[sl] = r
            return 0

        lax.fori_loop(0, mseg // _L, mrg, 0)
        pltpu.sync_copy(acc2_v.at[pl.ds(0, mseg)], d2o.at[b, pl.ds(m0, mseg)])

    return sc_kernel


def _make_sc1(N, M):
    n_chunk = N // (_NC * _NS)
    mseg = M // _NS
    f32 = jnp.float32
    mesh = plsc.VectorSubcoreMesh(core_axis_name="c", subcore_axis_name="s",
                                  num_cores=_NC, num_subcores=_NS)

    @functools.partial(
        pl.kernel, mesh=mesh,
        out_type=[jax.ShapeDtypeStruct((1, N), f32),
                  jax.ShapeDtypeStruct((_NC, M), f32)],
        scratch_types=[
            pltpu.VMEM((4, n_chunk), f32),
            pltpu.VMEM((4, M), f32),
            pltpu.VMEM((3, n_chunk), f32),
            pltpu.VMEM((3, M), f32),
            pltpu.VMEM((M,), f32),
            pltpu.VMEM((n_chunk,), f32),
            pltpu.VMEM((_NS, mseg), f32),
            pltpu.VMEM_SHARED((_NS, M), f32),
        ],
    )
    def sc_kernel(xt, yt, d1o, d2p, xb_v, y_v, xtmp_v, ytmp_v,
                  acc2_v, d1_v, mrg_v, sh_sp):
        c = lax.axis_index("c")
        s = lax.axis_index("s")
        q32 = s * _NC + c
        n0 = q32 * n_chunk

        pltpu.sync_copy(xt.at[0, :, pl.ds(n0, n_chunk)], xtmp_v)
        pltpu.sync_copy(yt.at[0], ytmp_v)

        def build_x(i, _):
            sl = pl.ds(i * _L, _L)
            x0 = xtmp_v[0, sl]
            x1 = xtmp_v[1, sl]
            x2c = xtmp_v[2, sl]
            xb_v[0, sl] = _bf16_round(x0) * -2.0
            xb_v[1, sl] = _bf16_round(x1) * -2.0
            xb_v[2, sl] = _bf16_round(x2c) * -2.0
            xb_v[3, sl] = x0 * x0 + x1 * x1 + x2c * x2c
            return 0

        lax.fori_loop(0, n_chunk // _L, build_x, 0)

        def build_y(i, _):
            sl = pl.ds(i * _L, _L)
            y0 = ytmp_v[0, sl]
            y1 = ytmp_v[1, sl]
            y2c = ytmp_v[2, sl]
            y_v[0, sl] = _bf16_round(y0)
            y_v[1, sl] = _bf16_round(y1)
            y_v[2, sl] = _bf16_round(y2c)
            y_v[3, sl] = y0 * y0 + y1 * y1 + y2c * y2c
            acc2_v[sl] = jnp.full((_L,), jnp.inf, f32)
            return 0

        lax.fori_loop(0, M // _L, build_y, 0)

        _sc_compute_rows(xb_v, y_v, acc2_v, d1_v, n_chunk, M)

        pltpu.sync_copy(d1_v, d1o.at[0, pl.ds(n0, n_chunk)])

        pltpu.sync_copy(acc2_v, sh_sp.at[s])
        plsc.subcore_barrier()
        m0 = s * mseg
        for k in range(_NS):
            pltpu.sync_copy(sh_sp.at[k, pl.ds(m0, mseg)], mrg_v.at[k])

        def mrg(i, _):
            sl = pl.ds(i * _L, _L)
            r = mrg_v[0, sl]
            for k in range(1, _NS):
                r = jnp.minimum(r, mrg_v[k, sl])
            acc2_v[sl] = r
            return 0

        lax.fori_loop(0, mseg // _L, mrg, 0)
        pltpu.sync_copy(acc2_v.at[pl.ds(0, mseg)],
                        d2p.at[c, pl.ds(m0, mseg)])

    return sc_kernel


def _make_sc1_merge(M):
    seg = M // (_NC * _NS)
    f32 = jnp.float32
    mesh = plsc.VectorSubcoreMesh(core_axis_name="c", subcore_axis_name="s",
                                  num_cores=_NC, num_subcores=_NS)

    @functools.partial(
        pl.kernel, mesh=mesh,
        out_type=jax.ShapeDtypeStruct((1, M), f32),
        scratch_types=[
            pltpu.VMEM((seg,), f32),
            pltpu.VMEM((seg,), f32),
        ],
    )
    def merge_kernel(d2p, out, a_v, b_v):
        c = lax.axis_index("c")
        s = lax.axis_index("s")
        m0 = (s * _NC + c) * seg
        pltpu.sync_copy(d2p.at[0, pl.ds(m0, seg)], a_v)
        pltpu.sync_copy(d2p.at[1, pl.ds(m0, seg)], b_v)

        def mrg(i, _):
            sl = pl.ds(i * _L, _L)
            a_v[sl] = jnp.minimum(a_v[sl], b_v[sl])
            return 0

        lax.fori_loop(0, seg // _L, mrg, 0)
        pltpu.sync_copy(a_v, out.at[0, pl.ds(m0, seg)])

    return merge_kernel


def _nnd_sc(x, y):
    B, N, _ = x.shape
    M = y.shape[1]
    xt = jnp.transpose(x, (0, 2, 1))
    yt = jnp.transpose(y, (0, 2, 1))
    if B == 1:
        d1, d2p = _make_sc1(N, M)(xt, yt)
        d2 = _make_sc1_merge(M)(d2p)
        return d1, d2
    d1, d2 = _make_sc(B, N, M)(xt, yt)
    return d1, d2


_B_SC = 0


def kernel(input1, input2):
    B = input1.shape[0]
    btc = B - _B_SC
    outs = []
    if btc > 0:
        outs.append(_nnd_tc(input1[:btc], input2[:btc]))
    if _B_SC > 0:
        outs.append(_nnd_sc(input1[btc:], input2[btc:]))
    if len(outs) == 1:
        return outs[0]
    return (jnp.concatenate([outs[0][0], outs[1][0]], axis=0),
            jnp.concatenate([outs[0][1], outs[1][1]], axis=0))


# speedup vs baseline: 1.8867x; 1.0403x over previous
import functools

import jax
import jax.numpy as jnp
from jax import lax
from jax.experimental import pallas as pl
from jax.experimental.pallas import tpu as pltpu
from jax.experimental.pallas import tpu_sc as plsc

_TN = 2048


def _nnd_body(x_ref, yt_ref, d1_ref, d2_ref):
    i = pl.program_id(1)
    x = x_ref[0]
    y = yt_ref[0]
    f32, bf16 = jnp.float32, jnp.bfloat16
    tn = x.shape[0]
    m = y.shape[1]
    x2 = (x[:, 0:1] * x[:, 0:1] + x[:, 1:2] * x[:, 1:2]
          + x[:, 2:3] * x[:, 2:3])
    y2 = (y[0:1, :] * y[0:1, :] + y[1:2, :] * y[1:2, :]
          + y[2:3, :] * y[2:3, :])
    x2h = x2.astype(bf16)
    x2l = (x2 - x2h.astype(f32)).astype(bf16)
    y2h = y2.astype(bf16)
    y2l = (y2 - y2h.astype(f32)).astype(bf16)
    one_c = jnp.ones((tn, 1), bf16)
    one_r = jnp.ones((1, m), bf16)
    xa = jnp.concatenate(
        [(x * -2.0).astype(bf16), one_c, one_c], axis=1)
    ya = jnp.concatenate(
        [y.astype(bf16), y2h, y2l], axis=0)
    nchunks = 4
    mc = m // nchunks
    d1p = []
    m2p = []
    for ci in range(nchunks):
        d = jax.lax.dot_general(xa, ya[:, ci * mc:(ci + 1) * mc],
                                (((1,), (0,)), ((), ())),
                                preferred_element_type=jnp.float32)
        d = d + x2
        d1p.append(jnp.min(d, axis=1, keepdims=True))
        m2p.append(jnp.min(d, axis=0, keepdims=True))
    d1c = d1p[0]
    for ci in range(1, nchunks):
        d1c = jnp.minimum(d1c, d1p[ci])
    d1_ref[0] = d1c
    m2 = jnp.concatenate(m2p, axis=1)

    @pl.when(i == 0)
    def _init():
        d2_ref[0] = m2

    @pl.when(i > 0)
    def _acc():
        d2_ref[0] = jnp.minimum(d2_ref[0], m2)


def _nnd_tc(x, y):
    B, N, _ = x.shape
    M = y.shape[1]
    yt = jnp.transpose(y, (0, 2, 1))
    d1, d2 = pl.pallas_call(
        _nnd_body,
        grid=(B, N // _TN),
        in_specs=[
            pl.BlockSpec((1, _TN, 3), lambda b, i: (b, i, 0)),
            pl.BlockSpec((1, 3, M), lambda b, i: (b, 0, 0)),
        ],
        out_specs=[
            pl.BlockSpec((1, _TN, 1), lambda b, i: (b, i, 0)),
            pl.BlockSpec((1, 1, M), lambda b, i: (b, 0, 0)),
        ],
        out_shape=[
            jax.ShapeDtypeStruct((B, N, 1), jnp.float32),
            jax.ShapeDtypeStruct((B, 1, M), jnp.float32),
        ],
    )(x, yt)
    return d1[:, :, 0], d2[:, 0, :]


_NC, _NS, _L = 2, 16, 16
_NBLK = 4
_UNROLL = 4


def _bf16_round(v):
    u = lax.bitcast_convert_type(v, jnp.uint32)
    r = (u + ((u >> 16) & jnp.uint32(1)) + jnp.uint32(0x7FFF)) \
        & jnp.uint32(0xFFFF0000)
    return lax.bitcast_convert_type(r, jnp.float32)


def _gather16(v, idx):
    return lax.gather(
        v, idx[:, None],
        lax.GatherDimensionNumbers(offset_dims=(), collapsed_slice_dims=(0,),
                                   start_index_map=(0,)),
        slice_sizes=(1,),
        mode=lax.GatherScatterMode.PROMISE_IN_BOUNDS)


def _sc_compute_rows(xb_v, y_v, acc2_v, d1_v, n_chunk, M):
    f32 = jnp.float32

    def n_block16(i, _):
        nb16 = i * _L
        sl16 = pl.ds(nb16, _L)
        a0v = xb_v[0, sl16]
        a1v = xb_v[1, sl16]
        a2v = xb_v[2, sl16]
        sxv = xb_v[3, sl16]
        lane = lax.iota(jnp.int32, _L)
        dv = jnp.full((_L,), 0.0, f32)
        for r in range(_L // _NBLK):
            coef = [(a0v[_NBLK * r + j], a1v[_NBLK * r + j],
                     a2v[_NBLK * r + j], sxv[_NBLK * r + j])
                    for j in range(_NBLK)]

            inf16 = jnp.full((_L,), jnp.inf, f32)

            @plsc.parallel_loop(0, M // _L, _UNROLL,
                                carry=tuple(inf16 for _ in range(_NBLK)))
            def accs(mj, accs_c):
                accs_c = list(accs_c)
                sls = [pl.ds((mj + u) * _L, _L) for u in range(_UNROLL)]
                ybs = [(y_v[0, sl], y_v[1, sl], y_v[2, sl], y_v[3, sl])
                       for sl in sls]
                a2s = [acc2_v[sl] for sl in sls]
                prods = [[(ybs[u][0] * coef[j][0],
                           ybs[u][1] * coef[j][1],
                           ybs[u][2] * coef[j][2])
                          for j in range(_NBLK)]
                         for u in range(_UNROLL)]
                ts = [[(ybs[u][3] + prods[u][j][2])
                       + (prods[u][j][0] + prods[u][j][1])
                       for j in range(_NBLK)]
                      for u in range(_UNROLL)]
                for j in range(_NBLK):
                    red = accs_c[j]
                    for u in range(_UNROLL):
                        red = jnp.minimum(red, ts[u][j])
                    accs_c[j] = red
                for u in range(_UNROLL):
                    dmins = [ts[u][j] + coef[j][3] for j in range(_NBLK)]
                    while len(dmins) > 1:
                        dmins = [jnp.minimum(dmins[k], dmins[k + 1])
                                 for k in range(0, len(dmins) - 1, 2)] \
                            + ([dmins[-1]] if len(dmins) % 2 else [])
                    acc2_v[sls[u]] = jnp.minimum(a2s[u], dmins[0])
                return tuple(accs_c)

            for j in range(_NBLK):
                red = accs[j]
                for sh in (8, 4, 2, 1):
                    pidx = jnp.bitwise_xor(lane, sh)
                    red = jnp.minimum(red, _gather16(red, pidx))
                dv = jnp.where(lane == (_NBLK * r + j), red, dv)
        d1_v[sl16] = dv + sxv
        return 0

    lax.fori_loop(0, n_chunk // _L, n_block16, 0)


def _make_sc(B, N, M):
    bps = B // _NC
    wpb = _NS // bps
    n_chunk = N // wpb
    mseg = M // wpb
    f32 = jnp.float32
    mesh = plsc.VectorSubcoreMesh(core_axis_name="c", subcore_axis_name="s",
                                  num_cores=_NC, num_subcores=_NS)

    @functools.partial(
        pl.kernel, mesh=mesh,
        out_type=[jax.ShapeDtypeStruct((B, N), f32),
                  jax.ShapeDtypeStruct((B, M), f32)],
        scratch_types=[
            pltpu.VMEM((4, n_chunk), f32),
            pltpu.VMEM((4, M), f32),
            pltpu.VMEM((3, n_chunk), f32),
            pltpu.VMEM((3, M), f32),
            pltpu.VMEM((M,), f32),
            pltpu.VMEM((n_chunk,), f32),
            pltpu.VMEM((wpb, mseg), f32),
            pltpu.VMEM_SHARED((_NS, M), f32),
        ],
    )
    def sc_kernel(xt, yt, d1o, d2o, xb_v, y_v, xtmp_v, ytmp_v,
                  acc2_v, d1_v, mrg_v, sh_sp):
        c = lax.axis_index("c")
        s = lax.axis_index("s")
        b = c * bps + s // wpb
        q = s % wpb
        n0 = q * n_chunk

        pltpu.sync_copy(xt.at[b, :, pl.ds(n0, n_chunk)], xtmp_v)
        pltpu.sync_copy(yt.at[b], ytmp_v)

        def build_x(i, _):
            sl = pl.ds(i * _L, _L)
            x0 = xtmp_v[0, sl]
            x1 = xtmp_v[1, sl]
            x2c = xtmp_v[2, sl]
            xb_v[0, sl] = _bf16_round(x0) * -2.0
            xb_v[1, sl] = _bf16_round(x1) * -2.0
            xb_v[2, sl] = _bf16_round(x2c) * -2.0
            xb_v[3, sl] = x0 * x0 + x1 * x1 + x2c * x2c
            return 0

        lax.fori_loop(0, n_chunk // _L, build_x, 0)

        def build_y(i, _):
            sl = pl.ds(i * _L, _L)
            y0 = ytmp_v[0, sl]
            y1 = ytmp_v[1, sl]
            y2c = ytmp_v[2, sl]
            y_v[0, sl] = _bf16_round(y0)
            y_v[1, sl] = _bf16_round(y1)
            y_v[2, sl] = _bf16_round(y2c)
            y_v[3, sl] = y0 * y0 + y1 * y1 + y2c * y2c
            acc2_v[sl] = jnp.full((_L,), jnp.inf, f32)
            return 0

        lax.fori_loop(0, M // _L, build_y, 0)

        _sc_compute_rows(xb_v, y_v, acc2_v, d1_v, n_chunk, M)

        pltpu.sync_copy(d1_v, d1o.at[b, pl.ds(n0, n_chunk)])

        pltpu.sync_copy(acc2_v, sh_sp.at[s])
        plsc.subcore_barrier()
        g0 = (s // wpb) * wpb
        m0 = q * mseg
        for k in range(wpb):
            pltpu.sync_copy(sh_sp.at[g0 + k, pl.ds(m0, mseg)], mrg_v.at[k])

        def mrg(i, _):
            sl = pl.ds(i * _L, _L)
            r = mrg_v[0, sl]
            for k in range(1, wpb):
                r = jnp.minimum(r, mrg_v[k, sl])
            acc2_v[sl] = r
            return 0

        lax.fori_loop(0, mseg // _L, mrg, 0)
        pltpu.sync_copy(acc2_v.at[pl.ds(0, mseg)], d2o.at[b, pl.ds(m0, mseg)])

    return sc_kernel


def _make_sc1(N, M):
    n_chunk = N // (_NC * _NS)
    mseg = M // _NS
    f32 = jnp.float32
    mesh = plsc.VectorSubcoreMesh(core_axis_name="c", subcore_axis_name="s",
                                  num_cores=_NC, num_subcores=_NS)

    @functools.partial(
        pl.kernel, mesh=mesh,
        out_type=[jax.ShapeDtypeStruct((1, N), f32),
                  jax.ShapeDtypeStruct((_NC, M), f32)],
        scratch_types=[
            pltpu.VMEM((4, n_chunk), f32),
            pltpu.VMEM((4, M), f32),
            pltpu.VMEM((3, n_chunk), f32),
            pltpu.VMEM((3, M), f32),
            pltpu.VMEM((M,), f32),
            pltpu.VMEM((n_chunk,), f32),
            pltpu.VMEM((_NS, mseg), f32),
            pltpu.VMEM_SHARED((_NS, M), f32),
        ],
    )
    def sc_kernel(xt, yt, d1o, d2p, xb_v, y_v, xtmp_v, ytmp_v,
                  acc2_v, d1_v, mrg_v, sh_sp):
        c = lax.axis_index("c")
        s = lax.axis_index("s")
        q32 = s * _NC + c
        n0 = q32 * n_chunk

        pltpu.sync_copy(xt.at[0, :, pl.ds(n0, n_chunk)], xtmp_v)
        pltpu.sync_copy(yt.at[0], ytmp_v)

        def build_x(i, _):
            sl = pl.ds(i * _L, _L)
            x0 = xtmp_v[0, sl]
            x1 = xtmp_v[1, sl]
            x2c = xtmp_v[2, sl]
            xb_v[0, sl] = _bf16_round(x0) * -2.0
            xb_v[1, sl] = _bf16_round(x1) * -2.0
            xb_v[2, sl] = _bf16_round(x2c) * -2.0
            xb_v[3, sl] = x0 * x0 + x1 * x1 + x2c * x2c
            return 0

        lax.fori_loop(0, n_chunk // _L, build_x, 0)

        def build_y(i, _):
            sl = pl.ds(i * _L, _L)
            y0 = ytmp_v[0, sl]
            y1 = ytmp_v[1, sl]
            y2c = ytmp_v[2, sl]
            y_v[0, sl] = _bf16_round(y0)
            y_v[1, sl] = _bf16_round(y1)
            y_v[2, sl] = _bf16_round(y2c)
            y_v[3, sl] = y0 * y0 + y1 * y1 + y2c * y2c
            acc2_v[sl] = jnp.full((_L,), jnp.inf, f32)
            return 0

        lax.fori_loop(0, M // _L, build_y, 0)

        _sc_compute_rows(xb_v, y_v, acc2_v, d1_v, n_chunk, M)

        pltpu.sync_copy(d1_v, d1o.at[0, pl.ds(n0, n_chunk)])

        pltpu.sync_copy(acc2_v, sh_sp.at[s])
        plsc.subcore_barrier()
        m0 = s * mseg
        for k in range(_NS):
            pltpu.sync_copy(sh_sp.at[k, pl.ds(m0, mseg)], mrg_v.at[k])

        def mrg(i, _):
            sl = pl.ds(i * _L, _L)
            r = mrg_v[0, sl]
            for k in range(1, _NS):
                r = jnp.minimum(r, mrg_v[k, sl])
            acc2_v[sl] = r
            return 0

        lax.fori_loop(0, mseg // _L, mrg, 0)
        pltpu.sync_copy(acc2_v.at[pl.ds(0, mseg)],
                        d2p.at[c, pl.ds(m0, mseg)])

    return sc_kernel


def _make_sc1_merge(M):
    seg = M // (_NC * _NS)
    f32 = jnp.float32
    mesh = plsc.VectorSubcoreMesh(core_axis_name="c", subcore_axis_name="s",
                                  num_cores=_NC, num_subcores=_NS)

    @functools.partial(
        pl.kernel, mesh=mesh,
        out_type=jax.ShapeDtypeStruct((1, M), f32),
        scratch_types=[
            pltpu.VMEM((seg,), f32),
            pltpu.VMEM((seg,), f32),
        ],
    )
    def merge_kernel(d2p, out, a_v, b_v):
        c = lax.axis_index("c")
        s = lax.axis_index("s")
        m0 = (s * _NC + c) * seg
        pltpu.sync_copy(d2p.at[0, pl.ds(m0, seg)], a_v)
        pltpu.sync_copy(d2p.at[1, pl.ds(m0, seg)], b_v)

        def mrg(i, _):
            sl = pl.ds(i * _L, _L)
            a_v[sl] = jnp.minimum(a_v[sl], b_v[sl])
            return 0

        lax.fori_loop(0, seg // _L, mrg, 0)
        pltpu.sync_copy(a_v, out.at[0, pl.ds(m0, seg)])

    return merge_kernel


def _nnd_sc(x, y):
    B, N, _ = x.shape
    M = y.shape[1]
    xt = jnp.transpose(x, (0, 2, 1))
    yt = jnp.transpose(y, (0, 2, 1))
    if B == 1:
        d1, d2p = _make_sc1(N, M)(xt, yt)
        d2 = _make_sc1_merge(M)(d2p)
        return d1, d2
    d1, d2 = _make_sc(B, N, M)(xt, yt)
    return d1, d2


_B_SC = 0


def kernel(input1, input2):
    B = input1.shape[0]
    btc = B - _B_SC
    outs = []
    if btc > 0:
        outs.append(_nnd_tc(input1[:btc], input2[:btc]))
    if _B_SC > 0:
        outs.append(_nnd_sc(input1[btc:], input2[btc:]))
    if len(outs) == 1:
        return outs[0]
    return (jnp.concatenate([outs[0][0], outs[1][0]], axis=0),
            jnp.concatenate([outs[0][1], outs[1][1]], axis=0))


# speedup vs baseline: 1.8871x; 1.0002x over previous
import functools

import jax
import jax.numpy as jnp
from jax import lax
from jax.experimental import pallas as pl
from jax.experimental.pallas import tpu as pltpu
from jax.experimental.pallas import tpu_sc as plsc

_TN = 2048


def _nnd_body(x_ref, yt_ref, d1_ref, d2_ref):
    i = pl.program_id(1)
    x = x_ref[0]
    y = yt_ref[0]
    f32, bf16 = jnp.float32, jnp.bfloat16
    tn = x.shape[0]
    m = y.shape[1]
    x2 = (x[:, 0:1] * x[:, 0:1] + x[:, 1:2] * x[:, 1:2]
          + x[:, 2:3] * x[:, 2:3])
    y2 = (y[0:1, :] * y[0:1, :] + y[1:2, :] * y[1:2, :]
          + y[2:3, :] * y[2:3, :])
    y2h = y2.astype(bf16)
    y2l = (y2 - y2h.astype(f32)).astype(bf16)
    one_c = jnp.ones((tn, 1), bf16)
    xa = jnp.concatenate(
        [(x * -2.0).astype(bf16), one_c, one_c], axis=1)
    ya = jnp.concatenate(
        [y.astype(bf16), y2h, y2l], axis=0)
    nchunks = 4
    mc = m // nchunks
    d1p = []
    m2p = []
    for ci in range(nchunks):
        d = jax.lax.dot_general(xa, ya[:, ci * mc:(ci + 1) * mc],
                                (((1,), (0,)), ((), ())),
                                preferred_element_type=jnp.float32)
        d = d + x2
        d1p.append(jnp.min(d, axis=1, keepdims=True))
        m2p.append(jnp.min(d, axis=0, keepdims=True))
    d1c = d1p[0]
    for ci in range(1, nchunks):
        d1c = jnp.minimum(d1c, d1p[ci])
    d1_ref[0] = d1c
    m2 = jnp.concatenate(m2p, axis=1)

    @pl.when(i == 0)
    def _init():
        d2_ref[0] = m2

    @pl.when(i > 0)
    def _acc():
        d2_ref[0] = jnp.minimum(d2_ref[0], m2)


def _nnd_tc(x, y):
    B, N, _ = x.shape
    M = y.shape[1]
    yt = jnp.transpose(y, (0, 2, 1))
    d1, d2 = pl.pallas_call(
        _nnd_body,
        grid=(B, N // _TN),
        in_specs=[
            pl.BlockSpec((1, _TN, 3), lambda b, i: (b, i, 0)),
            pl.BlockSpec((1, 3, M), lambda b, i: (b, 0, 0)),
        ],
        out_specs=[
            pl.BlockSpec((1, _TN, 1), lambda b, i: (b, i, 0)),
            pl.BlockSpec((1, 1, M), lambda b, i: (b, 0, 0)),
        ],
        out_shape=[
            jax.ShapeDtypeStruct((B, N, 1), jnp.float32),
            jax.ShapeDtypeStruct((B, 1, M), jnp.float32),
        ],
    )(x, yt)
    return d1[:, :, 0], d2[:, 0, :]


_NC, _NS, _L = 2, 16, 16
_NBLK = 4
_UNROLL = 4


def _bf16_round(v):
    u = lax.bitcast_convert_type(v, jnp.uint32)
    r = (u + ((u >> 16) & jnp.uint32(1)) + jnp.uint32(0x7FFF)) \
        & jnp.uint32(0xFFFF0000)
    return lax.bitcast_convert_type(r, jnp.float32)


def _gather16(v, idx):
    return lax.gather(
        v, idx[:, None],
        lax.GatherDimensionNumbers(offset_dims=(), collapsed_slice_dims=(0,),
                                   start_index_map=(0,)),
        slice_sizes=(1,),
        mode=lax.GatherScatterMode.PROMISE_IN_BOUNDS)


def _sc_compute_rows(xb_v, y_v, acc2_v, d1_v, n_chunk, M):
    f32 = jnp.float32

    def n_block16(i, _):
        nb16 = i * _L
        sl16 = pl.ds(nb16, _L)
        a0v = xb_v[0, sl16]
        a1v = xb_v[1, sl16]
        a2v = xb_v[2, sl16]
        sxv = xb_v[3, sl16]
        lane = lax.iota(jnp.int32, _L)
        dv = jnp.full((_L,), 0.0, f32)
        for r in range(_L // _NBLK):
            coef = [(a0v[_NBLK * r + j], a1v[_NBLK * r + j],
                     a2v[_NBLK * r + j], sxv[_NBLK * r + j])
                    for j in range(_NBLK)]

            inf16 = jnp.full((_L,), jnp.inf, f32)

            @plsc.parallel_loop(0, M // _L, _UNROLL,
                                carry=tuple(inf16 for _ in range(_NBLK)))
            def accs(mj, accs_c):
                accs_c = list(accs_c)
                sls = [pl.ds((mj + u) * _L, _L) for u in range(_UNROLL)]
                ybs = [(y_v[0, sl], y_v[1, sl], y_v[2, sl], y_v[3, sl])
                       for sl in sls]
                a2s = [acc2_v[sl] for sl in sls]
                prods = [[(ybs[u][0] * coef[j][0],
                           ybs[u][1] * coef[j][1],
                           ybs[u][2] * coef[j][2])
                          for j in range(_NBLK)]
                         for u in range(_UNROLL)]
                ts = [[(ybs[u][3] + prods[u][j][2])
                       + (prods[u][j][0] + prods[u][j][1])
                       for j in range(_NBLK)]
                      for u in range(_UNROLL)]
                for j in range(_NBLK):
                    red = accs_c[j]
                    for u in range(_UNROLL):
                        red = jnp.minimum(red, ts[u][j])
                    accs_c[j] = red
                for u in range(_UNROLL):
                    dmins = [ts[u][j] + coef[j][3] for j in range(_NBLK)]
                    while len(dmins) > 1:
                        dmins = [jnp.minimum(dmins[k], dmins[k + 1])
                                 for k in range(0, len(dmins) - 1, 2)] \
                            + ([dmins[-1]] if len(dmins) % 2 else [])
                    acc2_v[sls[u]] = jnp.minimum(a2s[u], dmins[0])
                return tuple(accs_c)

            for j in range(_NBLK):
                red = accs[j]
                for sh in (8, 4, 2, 1):
                    pidx = jnp.bitwise_xor(lane, sh)
                    red = jnp.minimum(red, _gather16(red, pidx))
                dv = jnp.where(lane == (_NBLK * r + j), red, dv)
        d1_v[sl16] = dv + sxv
        return 0

    lax.fori_loop(0, n_chunk // _L, n_block16, 0)


def _make_sc(B, N, M):
    bps = B // _NC
    wpb = _NS // bps
    n_chunk = N // wpb
    mseg = M // wpb
    f32 = jnp.float32
    mesh = plsc.VectorSubcoreMesh(core_axis_name="c", subcore_axis_name="s",
                                  num_cores=_NC, num_subcores=_NS)

    @functools.partial(
        pl.kernel, mesh=mesh,
        out_type=[jax.ShapeDtypeStruct((B, N), f32),
                  jax.ShapeDtypeStruct((B, M), f32)],
        scratch_types=[
            pltpu.VMEM((4, n_chunk), f32),
            pltpu.VMEM((4, M), f32),
            pltpu.VMEM((3, n_chunk), f32),
            pltpu.VMEM((3, M), f32),
            pltpu.VMEM((M,), f32),
            pltpu.VMEM((n_chunk,), f32),
            pltpu.VMEM((wpb, mseg), f32),
            pltpu.VMEM_SHARED((_NS, M), f32),
        ],
    )
    def sc_kernel(xt, yt, d1o, d2o, xb_v, y_v, xtmp_v, ytmp_v,
                  acc2_v, d1_v, mrg_v, sh_sp):
        c = lax.axis_index("c")
        s = lax.axis_index("s")
        b = c * bps + s // wpb
        q = s % wpb
        n0 = q * n_chunk

        pltpu.sync_copy(xt.at[b, :, pl.ds(n0, n_chunk)], xtmp_v)
        pltpu.sync_copy(yt.at[b], ytmp_v)

        def build_x(i, _):
            sl = pl.ds(i * _L, _L)
            x0 = xtmp_v[0, sl]
            x1 = xtmp_v[1, sl]
            x2c = xtmp_v[2, sl]
            xb_v[0, sl] = _bf16_round(x0) * -2.0
            xb_v[1, sl] = _bf16_round(x1) * -2.0
            xb_v[2, sl] = _bf16_round(x2c) * -2.0
            xb_v[3, sl] = x0 * x0 + x1 * x1 + x2c * x2c
            return 0

        lax.fori_loop(0, n_chunk // _L, build_x, 0)

        def build_y(i, _):
            sl = pl.ds(i * _L, _L)
            y0 = ytmp_v[0, sl]
            y1 = ytmp_v[1, sl]
            y2c = ytmp_v[2, sl]
            y_v[0, sl] = _bf16_round(y0)
            y_v[1, sl] = _bf16_round(y1)
            y_v[2, sl] = _bf16_round(y2c)
            y_v[3, sl] = y0 * y0 + y1 * y1 + y2c * y2c
            acc2_v[sl] = jnp.full((_L,), jnp.inf, f32)
            return 0

        lax.fori_loop(0, M // _L, build_y, 0)

        _sc_compute_rows(xb_v, y_v, acc2_v, d1_v, n_chunk, M)

        pltpu.sync_copy(d1_v, d1o.at[b, pl.ds(n0, n_chunk)])

        pltpu.sync_copy(acc2_v, sh_sp.at[s])
        plsc.subcore_barrier()
        g0 = (s // wpb) * wpb
        m0 = q * mseg
        for k in range(wpb):
            pltpu.sync_copy(sh_sp.at[g0 + k, pl.ds(m0, mseg)], mrg_v.at[k])

        def mrg(i, _):
            sl = pl.ds(i * _L, _L)
            r = mrg_v[0, sl]
            for k in range(1, wpb):
                r = jnp.minimum(r, mrg_v[k, sl])
            acc2_v[sl] = r
            return 0

        lax.fori_loop(0, mseg // _L, mrg, 0)
        pltpu.sync_copy(acc2_v.at[pl.ds(0, mseg)], d2o.at[b, pl.ds(m0, mseg)])

    return sc_kernel


def _make_sc1(N, M):
    n_chunk = N // (_NC * _NS)
    mseg = M // _NS
    f32 = jnp.float32
    mesh = plsc.VectorSubcoreMesh(core_axis_name="c", subcore_axis_name="s",
                                  num_cores=_NC, num_subcores=_NS)

    @functools.partial(
        pl.kernel, mesh=mesh,
        out_type=[jax.ShapeDtypeStruct((1, N), f32),
                  jax.ShapeDtypeStruct((_NC, M), f32)],
        scratch_types=[
            pltpu.VMEM((4, n_chunk), f32),
            pltpu.VMEM((4, M), f32),
            pltpu.VMEM((3, n_chunk), f32),
            pltpu.VMEM((3, M), f32),
            pltpu.VMEM((M,), f32),
            pltpu.VMEM((n_chunk,), f32),
            pltpu.VMEM((_NS, mseg), f32),
            pltpu.VMEM_SHARED((_NS, M), f32),
        ],
    )
    def sc_kernel(xt, yt, d1o, d2p, xb_v, y_v, xtmp_v, ytmp_v,
                  acc2_v, d1_v, mrg_v, sh_sp):
        c = lax.axis_index("c")
        s = lax.axis_index("s")
        q32 = s * _NC + c
        n0 = q32 * n_chunk

        pltpu.sync_copy(xt.at[0, :, pl.ds(n0, n_chunk)], xtmp_v)
        pltpu.sync_copy(yt.at[0], ytmp_v)

        def build_x(i, _):
            sl = pl.ds(i * _L, _L)
            x0 = xtmp_v[0, sl]
            x1 = xtmp_v[1, sl]
            x2c = xtmp_v[2, sl]
            xb_v[0, sl] = _bf16_round(x0) * -2.0
            xb_v[1, sl] = _bf16_round(x1) * -2.0
            xb_v[2, sl] = _bf16_round(x2c) * -2.0
            xb_v[3, sl] = x0 * x0 + x1 * x1 + x2c * x2c
            return 0

        lax.fori_loop(0, n_chunk // _L, build_x, 0)

        def build_y(i, _):
            sl = pl.ds(i * _L, _L)
            y0 = ytmp_v[0, sl]
            y1 = ytmp_v[1, sl]
            y2c = ytmp_v[2, sl]
            y_v[0, sl] = _bf16_round(y0)
            y_v[1, sl] = _bf16_round(y1)
            y_v[2, sl] = _bf16_round(y2c)
            y_v[3, sl] = y0 * y0 + y1 * y1 + y2c * y2c
            acc2_v[sl] = jnp.full((_L,), jnp.inf, f32)
            return 0

        lax.fori_loop(0, M // _L, build_y, 0)

        _sc_compute_rows(xb_v, y_v, acc2_v, d1_v, n_chunk, M)

        pltpu.sync_copy(d1_v, d1o.at[0, pl.ds(n0, n_chunk)])

        pltpu.sync_copy(acc2_v, sh_sp.at[s])
        plsc.subcore_barrier()
        m0 = s * mseg
        for k in range(_NS):
            pltpu.sync_copy(sh_sp.at[k, pl.ds(m0, mseg)], mrg_v.at[k])

        def mrg(i, _):
            sl = pl.ds(i * _L, _L)
            r = mrg_v[0, sl]
            for k in range(1, _NS):
                r = jnp.minimum(r, mrg_v[k, sl])
            acc2_v[sl] = r
            return 0

        lax.fori_loop(0, mseg // _L, mrg, 0)
        pltpu.sync_copy(acc2_v.at[pl.ds(0, mseg)],
                        d2p.at[c, pl.ds(m0, mseg)])

    return sc_kernel


def _make_sc1_merge(M):
    seg = M // (_NC * _NS)
    f32 = jnp.float32
    mesh = plsc.VectorSubcoreMesh(core_axis_name="c", subcore_axis_name="s",
                                  num_cores=_NC, num_subcores=_NS)

    @functools.partial(
        pl.kernel, mesh=mesh,
        out_type=jax.ShapeDtypeStruct((1, M), f32),
        scratch_types=[
            pltpu.VMEM((seg,), f32),
            pltpu.VMEM((seg,), f32),
        ],
    )
    def merge_kernel(d2p, out, a_v, b_v):
        c = lax.axis_index("c")
        s = lax.axis_index("s")
        m0 = (s * _NC + c) * seg
        pltpu.sync_copy(d2p.at[0, pl.ds(m0, seg)], a_v)
        pltpu.sync_copy(d2p.at[1, pl.ds(m0, seg)], b_v)

        def mrg(i, _):
            sl = pl.ds(i * _L, _L)
            a_v[sl] = jnp.minimum(a_v[sl], b_v[sl])
            return 0

        lax.fori_loop(0, seg // _L, mrg, 0)
        pltpu.sync_copy(a_v, out.at[0, pl.ds(m0, seg)])

    return merge_kernel


def _nnd_sc(x, y):
    B, N, _ = x.shape
    M = y.shape[1]
    xt = jnp.transpose(x, (0, 2, 1))
    yt = jnp.transpose(y, (0, 2, 1))
    if B == 1:
        d1, d2p = _make_sc1(N, M)(xt, yt)
        d2 = _make_sc1_merge(M)(d2p)
        return d1, d2
    d1, d2 = _make_sc(B, N, M)(xt, yt)
    return d1, d2


_B_SC = 0


def kernel(input1, input2):
    B = input1.shape[0]
    btc = B - _B_SC
    outs = []
    if btc > 0:
        outs.append(_nnd_tc(input1[:btc], input2[:btc]))
    if _B_SC > 0:
        outs.append(_nnd_sc(input1[btc:], input2[btc:]))
    if len(outs) == 1:
        return outs[0]
    return (jnp.concatenate([outs[0][0], outs[1][0]], axis=0),
            jnp.concatenate([outs[0][1], outs[1][1]], axis=0))
